# baseline (speedup 1.0000x reference)
import math

import jax
import jax.numpy as jnp
from jax import lax
from jax.experimental import pallas as pl
from jax.experimental.pallas import tpu as pltpu
from jax.experimental.pallas import tpu_sc as plsc

N = 10000
E = 320000
H = 128
EF = 20
ET = 3
B = 16
NC = 20
NCAND = B * NC
NCORES = 2
NSUB = 16
NW = NCORES * NSUB
EPW = E // NW
CH = 48
HX = 256
UN = 5
ISQ = 1.0 / math.sqrt(H)


def _proj_body(x_ref, wl_ref, bl_ref, wr_ref, br_ref, embp_ref, wep_ref,
               be_ref, qx_ref, k_ref):
    x = x_ref[...]
    dn = (((1,), (1,)), ((), ()))
    q = lax.dot_general(x, wl_ref[...], dn, preferred_element_type=jnp.float32)
    q = (q + bl_ref[...]) * ISQ
    qx_ref[:, 0:H] = q
    k = lax.dot_general(x, wr_ref[...], dn, preferred_element_type=jnp.float32)
    k_ref[...] = k + br_ref[...]
    ef = jax.nn.gelu(embp_ref[...])
    k2 = lax.dot_general(ef, wep_ref[...], dn,
                         preferred_element_type=jnp.float32) + be_ref[...]
    q2 = lax.dot_general(q, k2, (((1,), (1,)), ((), ())),
                         precision=lax.Precision.HIGHEST,
                         preferred_element_type=jnp.float32)
    qx_ref[:, H:H + 8] = q2


def _tc_proj(h, Wl, bl, Wr, br, emb_p, We_p, be):
    grid = 10
    rows = N // grid
    return pl.pallas_call(
        _proj_body,
        grid=(grid,),
        in_specs=[
            pl.BlockSpec((rows, H), lambda i: (i, 0)),
            pl.BlockSpec((H, H), lambda i: (0, 0)),
            pl.BlockSpec((1, H), lambda i: (0, 0)),
            pl.BlockSpec((H, H), lambda i: (0, 0)),
            pl.BlockSpec((1, H), lambda i: (0, 0)),
            pl.BlockSpec((8, 32), lambda i: (0, 0)),
            pl.BlockSpec((H, 32), lambda i: (0, 0)),
            pl.BlockSpec((1, H), lambda i: (0, 0)),
        ],
        out_specs=[
            pl.BlockSpec((rows, HX), lambda i: (i, 0)),
            pl.BlockSpec((rows, H), lambda i: (i, 0)),
        ],
        out_shape=[
            jax.ShapeDtypeStruct((N, HX), jnp.float32),
            jax.ShapeDtypeStruct((N, H), jnp.float32),
        ],
    )(h, Wl, bl, Wr, br, emb_p, We_p, be)


def _sc_body(dst_h, src_h, et_h, cand_h, scores_h, q_h, k1_h,
             s_out, w_out, smap_out,
             dst_sh, src_sh, et_sh, slot_t, scores_v, cand_v,
             srcc, dstc, etc_, qrows, k1rows,
             s_v, w_v, smap_v, sem_a, sem_s,
             sem_q0, sem_q1, sem_k0, sem_k1):
    wid = lax.axis_index("s") * NCORES + lax.axis_index("c")
    lane = lax.iota(jnp.int32, 16)
    zeros_i = jnp.zeros((16,), jnp.int32)
    zeros_f = jnp.zeros((16,), jnp.float32)
    base = wid * EPW

    cp_d = pltpu.make_async_copy(dst_h.at[pl.ds(base, EPW)], dst_sh, sem_a)
    cp_s = pltpu.make_async_copy(src_h.at[pl.ds(base, EPW)], src_sh, sem_a)
    cp_e = pltpu.make_async_copy(et_h.at[pl.ds(base, EPW)], et_sh, sem_a)
    cp_sc = pltpu.make_async_copy(scores_h, scores_v, sem_s)
    cp_d.start()
    cp_s.start()
    cp_e.start()
    cp_sc.start()
    pltpu.sync_copy(cand_h, cand_v)

    neg1 = jnp.full((16,), -1, jnp.int32)

    def init_body(i, _):
        for u in range(UN):
            slot_t[pl.ds((i * UN + u) * 16, 16)] = neg1
        return 0

    lax.fori_loop(0, N // (16 * UN), init_body, 0)

    def zacc_body(i, _):
        s_v[i] = zeros_f
        w_v[i] = zeros_f
        return 0

    lax.fori_loop(0, NCAND // 16, zacc_body, 0)

    def slot_body(i, _):
        c16 = cand_v[pl.ds(i * 16, 16)]
        plsc.store_scatter(slot_t, [c16], i * 16 + lane)
        return 0

    lax.fori_loop(0, NCAND // 16, slot_body, 0)

    cp_d.wait()
    cp_s.wait()
    cp_e.wait()

    def j_body(j, off_v):
        for u in range(UN):
            sl = pl.ds((j * UN + u) * 16, 16)
            d = dst_sh[sl]
            s = plsc.load_gather(slot_t, [d])
            m = s >= 0
            pos = off_v + plsc.cumsum(jnp.where(m, 1, 0)) - 1
            plsc.store_scatter(srcc, [pos], src_sh[sl], mask=m)
            plsc.store_scatter(dstc, [pos], d, mask=m)
            plsc.store_scatter(etc_, [pos], et_sh[sl], mask=m)
            off_v = off_v + plsc.all_reduce_population_count(m)
        return off_v

    off_v = lax.fori_loop(0, EPW // (16 * UN), j_body,
                          jnp.zeros((16,), jnp.int32))
    cnt = off_v[0]

    def tail_body(i, _):
        sl = pl.ds(cnt + i * 16, 16)
        srcc[sl] = zeros_i
        dstc[sl] = zeros_i
        etc_[sl] = zeros_i
        return 0

    lax.fori_loop(0, CH // 16, tail_body, 0)

    cp_sc.wait()

    nch = (cnt + CH - 1) // CH

    def start_chunk(g, sq, sk, slot):
        cq = pltpu.make_async_copy(q_h.at[dstc.at[pl.ds(g * CH, CH)]],
                                   qrows.at[pl.ds(slot * CH, CH)], sq)
        ck = pltpu.make_async_copy(k1_h.at[srcc.at[pl.ds(g * CH, CH)]],
                                   k1rows.at[pl.ds(slot * CH, CH)], sk)
        cq.start()
        ck.start()

    def wait_chunk(g, sq, sk, slot):
        pltpu.make_async_copy(q_h.at[dstc.at[pl.ds(g * CH, CH)]],
                              qrows.at[pl.ds(slot * CH, CH)], sq).wait()
        pltpu.make_async_copy(k1_h.at[srcc.at[pl.ds(g * CH, CH)]],
                              k1rows.at[pl.ds(slot * CH, CH)], sk).wait()

    @pl.when(nch > 0)
    def _():
        start_chunk(0, sem_q0, sem_k0, 0)

    @pl.when(nch > 1)
    def _():
        start_chunk(1, sem_q1, sem_k1, 1)

    def g_body(g, _):
        slot = lax.rem(g, 2)

        @pl.when(slot == 0)
        def _():
            wait_chunk(g, sem_q0, sem_k0, 0)

        @pl.when(slot == 1)
        def _():
            wait_chunk(g, sem_q1, sem_k1, 1)

        def grp_body(gg, _):
            eb = g * CH + gg * 16
            sl = pl.ds(eb, 16)
            et16 = etc_[sl]
            s16 = srcc[sl]
            slot16 = plsc.load_gather(slot_t, [dstc[sl]])
            m = (eb + lane) < cnt
            row16 = slot * CH + gg * 16 + lane
            acc0 = plsc.load_gather(qrows, [row16, et16 + H])

            def c_body(c8, acc):
                for u in range(8):
                    cvec = zeros_i + (c8 * 8 + u)
                    qc = plsc.load_gather(qrows, [row16, cvec])
                    kc = plsc.load_gather(k1rows, [row16, cvec])
                    acc = acc + qc * kc
                return acc

            acc = lax.fori_loop(0, H // 8, c_body, acc0)
            ex = jnp.where(m, jnp.exp(acc), 0.0)
            ss = plsc.load_gather(scores_v, [s16])
            hi = lax.shift_right_logical(slot16, 4)
            lo = lax.bitwise_and(slot16, 15)
            plsc.addupdate_scatter(s_v, [hi, lo], ex, mask=m)
            plsc.addupdate_scatter(w_v, [hi, lo], ss * ex, mask=m)
            return 0

        lax.fori_loop(0, CH // 16, grp_body, 0)

        @pl.when(g + 2 < nch)
        def _():

            @pl.when(slot == 0)
            def _():
                start_chunk(g + 2, sem_q0, sem_k0, 0)

            @pl.when(slot == 1)
            def _():
                start_chunk(g + 2, sem_q1, sem_k1, 1)

        return 0

    lax.fori_loop(0, nch, g_body, 0)

    pltpu.sync_copy(s_v, s_out.at[wid])
    pltpu.sync_copy(w_v, w_out.at[wid])

    @pl.when(wid == 0)
    def _():
        def sm_body(i, _):
            c16 = cand_v[pl.ds(i * 16, 16)]
            smap_v[pl.ds(i * 16, 16)] = plsc.load_gather(slot_t, [c16])
            return 0

        lax.fori_loop(0, NCAND // 16, sm_body, 0)
        pltpu.sync_copy(smap_v, smap_out)


def _sc_main(dst, src, et, cand_flat, scores_flat, qx, k1):
    mesh = plsc.VectorSubcoreMesh(core_axis_name="c", subcore_axis_name="s",
                                  num_cores=NCORES, num_subcores=NSUB)
    f = pl.kernel(
        _sc_body,
        out_type=[
            jax.ShapeDtypeStruct((NW, NCAND // 16, 16), jnp.float32),
            jax.ShapeDtypeStruct((NW, NCAND // 16, 16), jnp.float32),
            jax.ShapeDtypeStruct((NCAND,), jnp.int32),
        ],
        mesh=mesh,
        scratch_types=[
            pltpu.VMEM((EPW,), jnp.int32),
            pltpu.VMEM((EPW,), jnp.int32),
            pltpu.VMEM((EPW,), jnp.int32),
            pltpu.VMEM((N,), jnp.int32),
            pltpu.VMEM((N,), jnp.float32),
            pltpu.VMEM((NCAND,), jnp.int32),
            pltpu.VMEM((EPW + CH,), jnp.int32),
            pltpu.VMEM((EPW + CH,), jnp.int32),
            pltpu.VMEM((EPW + CH,), jnp.int32),
            pltpu.VMEM((2 * CH, HX), jnp.float32),
            pltpu.VMEM((2 * CH, H), jnp.float32),
            pltpu.VMEM((NCAND // 16, 16), jnp.float32),
            pltpu.VMEM((NCAND // 16, 16), jnp.float32),
            pltpu.VMEM((NCAND,), jnp.int32),
            pltpu.SemaphoreType.DMA,
            pltpu.SemaphoreType.DMA,
            pltpu.SemaphoreType.DMA,
            pltpu.SemaphoreType.DMA,
            pltpu.SemaphoreType.DMA,
            pltpu.SemaphoreType.DMA,
        ],
        compiler_params=pltpu.CompilerParams(needs_layout_passes=False),
    )
    return f(dst, src, et, cand_flat, scores_flat, qx, k1)


def _combine_body(sp_ref, wp_ref, smap_ref, lab_ref, lf_ref, loss_ref):
    s = jnp.sum(sp_ref[...], axis=0, keepdims=True)
    w = jnp.sum(wp_ref[...], axis=0, keepdims=True)
    safe = jnp.where(s > 0, s, 1.0)
    upd = jnp.where(s > 0, w / safe, 0.0)
    ids = lax.broadcasted_iota(jnp.int32, (NCAND, NCAND), 1)
    oh = (smap_ref[...] == ids).astype(jnp.float32)
    xf = lax.dot_general(upd, oh, (((1,), (1,)), ((), ())),
                         precision=lax.Precision.HIGHEST,
                         preferred_element_type=jnp.float32)
    lf_ref[...] = xf

    gids = lax.broadcasted_iota(jnp.int32, (B, NCAND), 1) // NC
    rows = lax.broadcasted_iota(jnp.int32, (B, NCAND), 0)
    gmask = gids == rows
    xb = jnp.broadcast_to(xf, (B, NCAND))
    x2 = jnp.where(gmask, xb, -1e30)
    m = jnp.max(x2, axis=1, keepdims=True)
    lse = jnp.log(jnp.sum(jnp.exp(x2 - m), axis=1, keepdims=True)) + m
    cids = lax.broadcasted_iota(jnp.int32, (B, NCAND), 1) % NC
    pick = gmask & (cids == lab_ref[...])
    picked = jnp.sum(jnp.where(pick, xb, 0.0), axis=1, keepdims=True)
    loss_ref[...] = jnp.sum(lse - picked).reshape(1, 1)


def _tc_combine(s_part, w_part, smap, label):
    return pl.pallas_call(
        _combine_body,
        out_shape=[
            jax.ShapeDtypeStruct((1, NCAND), jnp.float32),
            jax.ShapeDtypeStruct((1, 1), jnp.float32),
        ],
    )(s_part, w_part, smap, label.reshape(B, 1))


def kernel(embedding, scores, edge_index, edge_type, cand_idx, label,
           Wl, bl, Wr, br, We, be, emb_table):
    emb_p = jnp.zeros((8, 32), jnp.float32).at[:ET, :EF].set(emb_table)
    We_p = jnp.zeros((H, 32), jnp.float32).at[:, :EF].set(We)
    qx, k1 = _tc_proj(embedding, Wl, bl.reshape(1, H), Wr,
                      br.reshape(1, H), emb_p, We_p, be.reshape(1, H))

    src = edge_index[0]
    dst = edge_index[1]
    cand_flat = cand_idx.reshape(NCAND)
    scores_flat = scores.reshape(N)

    s_part, w_part, smap = _sc_main(dst, src, edge_type, cand_flat,
                                    scores_flat, qx, k1)

    logits_flat, loss = _tc_combine(s_part.reshape(NW, NCAND),
                                    w_part.reshape(NW, NCAND),
                                    smap.reshape(NCAND, 1), label)
    logits = logits_flat.reshape(B, NC)
    return loss[0, 0], logits


# speedup vs baseline: 1.0848x; 1.0848x over previous
import math

import jax
import jax.numpy as jnp
from jax import lax
from jax.experimental import pallas as pl
from jax.experimental.pallas import tpu as pltpu
from jax.experimental.pallas import tpu_sc as plsc

N = 10000
E = 320000
H = 128
EF = 20
ET = 3
B = 16
NC = 20
NCAND = B * NC
NCORES = 2
NSUB = 16
NW = NCORES * NSUB
EPW = E // NW
CH = 32
UN = 5
ISQ = 1.0 / math.sqrt(H)


def _proj_body(x_ref, wl_ref, bl_ref, wr_ref, br_ref, embp_ref, wep_ref,
               be_ref, q_ref, k_ref, k2_ref):
    x = x_ref[...]
    dn = (((1,), (1,)), ((), ()))
    q = lax.dot_general(x, wl_ref[...], dn, preferred_element_type=jnp.float32)
    q_ref[...] = (q + bl_ref[...]) * ISQ
    k = lax.dot_general(x, wr_ref[...], dn, preferred_element_type=jnp.float32)
    k_ref[...] = k + br_ref[...]

    @pl.when(pl.program_id(0) == 0)
    def _():
        ef = jax.nn.gelu(embp_ref[...])
        k2 = lax.dot_general(ef, wep_ref[...], dn,
                             preferred_element_type=jnp.float32)
        k2_ref[...] = k2 + be_ref[...]


def _tc_proj(h, Wl, bl, Wr, br, emb_p, We_p, be):
    grid = 10
    rows = N // grid
    return pl.pallas_call(
        _proj_body,
        grid=(grid,),
        in_specs=[
            pl.BlockSpec((rows, H), lambda i: (i, 0)),
            pl.BlockSpec((H, H), lambda i: (0, 0)),
            pl.BlockSpec((1, H), lambda i: (0, 0)),
            pl.BlockSpec((H, H), lambda i: (0, 0)),
            pl.BlockSpec((1, H), lambda i: (0, 0)),
            pl.BlockSpec((8, 32), lambda i: (0, 0)),
            pl.BlockSpec((H, 32), lambda i: (0, 0)),
            pl.BlockSpec((1, H), lambda i: (0, 0)),
        ],
        out_specs=[
            pl.BlockSpec((rows, H), lambda i: (i, 0)),
            pl.BlockSpec((rows, H), lambda i: (i, 0)),
            pl.BlockSpec((8, H), lambda i: (0, 0)),
        ],
        out_shape=[
            jax.ShapeDtypeStruct((N, H), jnp.float32),
            jax.ShapeDtypeStruct((N, H), jnp.float32),
            jax.ShapeDtypeStruct((8, H), jnp.float32),
        ],
    )(h, Wl, bl, Wr, br, emb_p, We_p, be)


def _sc_body(dst_h, src_h, et_h, cand_h, scores_h, q_h, k1_h, k2_h,
             s_out, w_out, smap_out,
             dst_sh, src_sh, et_sh, slot_t, scores_v, cand_v, k2_v, q_cand,
             srcc, sec, k1rows,
             s_v, w_v, smap_v, sem_a, sem_s, sem_q, sem_k0, sem_k1):
    wid = lax.axis_index("s") * NCORES + lax.axis_index("c")
    lane = lax.iota(jnp.int32, 16)
    zeros_i = jnp.zeros((16,), jnp.int32)
    zeros_f = jnp.zeros((16,), jnp.float32)
    base = wid * EPW

    cp_d = pltpu.make_async_copy(dst_h.at[pl.ds(base, EPW)], dst_sh, sem_a)
    cp_s = pltpu.make_async_copy(src_h.at[pl.ds(base, EPW)], src_sh, sem_a)
    cp_e = pltpu.make_async_copy(et_h.at[pl.ds(base, EPW)], et_sh, sem_a)
    cp_sc = pltpu.make_async_copy(scores_h, scores_v, sem_s)
    cp_d.start()
    cp_s.start()
    cp_e.start()
    cp_sc.start()
    pltpu.sync_copy(cand_h, cand_v)
    pltpu.sync_copy(k2_h.at[pl.ds(0, 4)], k2_v)

    cq0 = pltpu.make_async_copy(q_h.at[cand_v.at[pl.ds(0, 128)]],
                                q_cand.at[pl.ds(0, 128)], sem_q)
    cq1 = pltpu.make_async_copy(q_h.at[cand_v.at[pl.ds(128, 128)]],
                                q_cand.at[pl.ds(128, 128)], sem_q)
    cq2 = pltpu.make_async_copy(q_h.at[cand_v.at[pl.ds(256, 64)]],
                                q_cand.at[pl.ds(256, 64)], sem_q)
    cq0.start()
    cq1.start()
    cq2.start()

    neg1 = jnp.full((16,), -1, jnp.int32)

    def init_body(i, _):
        for u in range(UN):
            slot_t[pl.ds((i * UN + u) * 16, 16)] = neg1
        return 0

    lax.fori_loop(0, N // (16 * UN), init_body, 0)

    def zacc_body(i, _):
        s_v[i] = zeros_f
        w_v[i] = zeros_f
        return 0

    lax.fori_loop(0, NCAND // 16, zacc_body, 0)

    def slot_body(i, _):
        c16 = cand_v[pl.ds(i * 16, 16)]
        plsc.store_scatter(slot_t, [c16], i * 16 + lane)
        return 0

    lax.fori_loop(0, NCAND // 16, slot_body, 0)

    cp_d.wait()
    cp_s.wait()
    cp_e.wait()

    def j_body(j, off_v):
        for u in range(UN):
            sl = pl.ds((j * UN + u) * 16, 16)
            d = dst_sh[sl]
            s = plsc.load_gather(slot_t, [d])
            m = s >= 0
            pos = off_v + plsc.cumsum(jnp.where(m, 1, 0)) - 1
            se = s + lax.shift_left(et_sh[sl], 9)
            plsc.store_scatter(srcc, [pos], src_sh[sl], mask=m)
            plsc.store_scatter(sec, [pos], se, mask=m)
            off_v = off_v + plsc.all_reduce_population_count(m)
        return off_v

    off_v = lax.fori_loop(0, EPW // (16 * UN), j_body,
                          jnp.zeros((16,), jnp.int32))
    cnt = off_v[0]

    def tail_body(i, _):
        sl = pl.ds(cnt + i * 16, 16)
        srcc[sl] = zeros_i
        sec[sl] = zeros_i
        return 0

    lax.fori_loop(0, CH // 16, tail_body, 0)

    cp_sc.wait()
    cq0.wait()
    cq1.wait()
    cq2.wait()

    nch = (cnt + CH - 1) // CH

    def start_chunk(g, sk, par):
        pltpu.make_async_copy(k1_h.at[srcc.at[pl.ds(g * CH, CH)]],
                              k1rows.at[pl.ds(par * CH, CH)], sk).start()

    def wait_chunk(g, sk, par):
        pltpu.make_async_copy(k1_h.at[srcc.at[pl.ds(g * CH, CH)]],
                              k1rows.at[pl.ds(par * CH, CH)], sk).wait()

    @pl.when(nch > 0)
    def _():
        start_chunk(0, sem_k0, 0)

    @pl.when(nch > 1)
    def _():
        start_chunk(1, sem_k1, 1)

    def g_body(g, _):
        par = lax.rem(g, 2)

        @pl.when(par == 0)
        def _():
            wait_chunk(g, sem_k0, 0)

        @pl.when(par == 1)
        def _():
            wait_chunk(g, sem_k1, 1)

        def grp_body(gg, _):
            eb = g * CH + gg * 16
            sl = pl.ds(eb, 16)
            se16 = sec[sl]
            s16 = srcc[sl]
            slot16 = lax.bitwise_and(se16, 511)
            et16 = lax.shift_right_logical(se16, 9)
            m = (eb + lane) < cnt
            row16 = par * CH + gg * 16 + lane

            def c_body(c8, acc):
                for u in range(8):
                    cvec = zeros_i + (c8 * 8 + u)
                    qc = plsc.load_gather(q_cand, [slot16, cvec])
                    kc = plsc.load_gather(k1rows, [row16, cvec])
                    k2c = plsc.load_gather(k2_v, [et16, cvec])
                    acc = acc + qc * (kc + k2c)
                return acc

            acc = lax.fori_loop(0, H // 8, c_body, zeros_f)
            ex = jnp.where(m, jnp.exp(acc), 0.0)
            ss = plsc.load_gather(scores_v, [s16])
            hi = lax.shift_right_logical(slot16, 4)
            lo = lax.bitwise_and(slot16, 15)
            plsc.addupdate_scatter(s_v, [hi, lo], ex, mask=m)
            plsc.addupdate_scatter(w_v, [hi, lo], ss * ex, mask=m)
            return 0

        lax.fori_loop(0, CH // 16, grp_body, 0)

        @pl.when(g + 2 < nch)
        def _():

            @pl.when(par == 0)
            def _():
                start_chunk(g + 2, sem_k0, 0)

            @pl.when(par == 1)
            def _():
                start_chunk(g + 2, sem_k1, 1)

        return 0

    lax.fori_loop(0, nch, g_body, 0)

    pltpu.sync_copy(s_v, s_out.at[wid])
    pltpu.sync_copy(w_v, w_out.at[wid])

    @pl.when(wid == 0)
    def _():
        def sm_body(i, _):
            c16 = cand_v[pl.ds(i * 16, 16)]
            smap_v[pl.ds(i * 16, 16)] = plsc.load_gather(slot_t, [c16])
            return 0

        lax.fori_loop(0, NCAND // 16, sm_body, 0)
        pltpu.sync_copy(smap_v, smap_out)


def _sc_main(dst, src, et, cand_flat, scores_flat, q, k1, k2p):
    mesh = plsc.VectorSubcoreMesh(core_axis_name="c", subcore_axis_name="s",
                                  num_cores=NCORES, num_subcores=NSUB)
    f = pl.kernel(
        _sc_body,
        out_type=[
            jax.ShapeDtypeStruct((NW, NCAND // 16, 16), jnp.float32),
            jax.ShapeDtypeStruct((NW, NCAND // 16, 16), jnp.float32),
            jax.ShapeDtypeStruct((NCAND,), jnp.int32),
        ],
        mesh=mesh,
        scratch_types=[
            pltpu.VMEM((EPW,), jnp.int32),
            pltpu.VMEM((EPW,), jnp.int32),
            pltpu.VMEM((EPW,), jnp.int32),
            pltpu.VMEM((N,), jnp.int32),
            pltpu.VMEM((N,), jnp.float32),
            pltpu.VMEM((NCAND,), jnp.int32),
            pltpu.VMEM((4, H), jnp.float32),
            pltpu.VMEM((NCAND, H), jnp.float32),
            pltpu.VMEM((EPW + CH,), jnp.int32),
            pltpu.VMEM((EPW + CH,), jnp.int32),
            pltpu.VMEM((2 * CH, H), jnp.float32),
            pltpu.VMEM((NCAND // 16, 16), jnp.float32),
            pltpu.VMEM((NCAND // 16, 16), jnp.float32),
            pltpu.VMEM((NCAND,), jnp.int32),
            pltpu.SemaphoreType.DMA,
            pltpu.SemaphoreType.DMA,
            pltpu.SemaphoreType.DMA,
            pltpu.SemaphoreType.DMA,
            pltpu.SemaphoreType.DMA,
        ],
        compiler_params=pltpu.CompilerParams(needs_layout_passes=False),
    )
    return f(dst, src, et, cand_flat, scores_flat, q, k1, k2p)


def _combine_body(sp_ref, wp_ref, smap_ref, lab_ref, lf_ref, loss_ref):
    s = jnp.sum(sp_ref[...], axis=0, keepdims=True)
    w = jnp.sum(wp_ref[...], axis=0, keepdims=True)
    safe = jnp.where(s > 0, s, 1.0)
    upd = jnp.where(s > 0, w / safe, 0.0)
    ids = lax.broadcasted_iota(jnp.int32, (NCAND, NCAND), 1)
    oh = (smap_ref[...] == ids).astype(jnp.float32)
    xf = lax.dot_general(upd, oh, (((1,), (1,)), ((), ())),
                         precision=lax.Precision.HIGHEST,
                         preferred_element_type=jnp.float32)
    lf_ref[...] = xf

    gids = lax.broadcasted_iota(jnp.int32, (B, NCAND), 1) // NC
    rows = lax.broadcasted_iota(jnp.int32, (B, NCAND), 0)
    gmask = gids == rows
    xb = jnp.broadcast_to(xf, (B, NCAND))
    x2 = jnp.where(gmask, xb, -1e30)
    m = jnp.max(x2, axis=1, keepdims=True)
    lse = jnp.log(jnp.sum(jnp.exp(x2 - m), axis=1, keepdims=True)) + m
    cids = lax.broadcasted_iota(jnp.int32, (B, NCAND), 1) % NC
    pick = gmask & (cids == lab_ref[...])
    picked = jnp.sum(jnp.where(pick, xb, 0.0), axis=1, keepdims=True)
    loss_ref[...] = jnp.sum(lse - picked).reshape(1, 1)


def _tc_combine(s_part, w_part, smap, label):
    return pl.pallas_call(
        _combine_body,
        out_shape=[
            jax.ShapeDtypeStruct((1, NCAND), jnp.float32),
            jax.ShapeDtypeStruct((1, 1), jnp.float32),
        ],
    )(s_part, w_part, smap, label.reshape(B, 1))


def kernel(embedding, scores, edge_index, edge_type, cand_idx, label,
           Wl, bl, Wr, br, We, be, emb_table):
    emb_p = jnp.zeros((8, 32), jnp.float32).at[:ET, :EF].set(emb_table)
    We_p = jnp.zeros((H, 32), jnp.float32).at[:, :EF].set(We)
    q, k1, k2p = _tc_proj(embedding, Wl, bl.reshape(1, H), Wr,
                          br.reshape(1, H), emb_p, We_p, be.reshape(1, H))

    src = edge_index[0]
    dst = edge_index[1]
    cand_flat = cand_idx.reshape(NCAND)
    scores_flat = scores.reshape(N)

    s_part, w_part, smap = _sc_main(dst, src, edge_type, cand_flat,
                                    scores_flat, q, k1, k2p)

    logits_flat, loss = _tc_combine(s_part.reshape(NW, NCAND),
                                    w_part.reshape(NW, NCAND),
                                    smap.reshape(NCAND, 1), label)
    logits = logits_flat.reshape(B, NC)
    return loss[0, 0], logits
